# Initial kernel scaffold; baseline (speedup 1.0000x reference)
import jax
import jax.numpy as jnp
from jax.experimental import pallas as pl


def kernel(x, edge_index, W1, b1, W2, b2, W3, b3, Wf1, bf1, Wf2, bf2):
    raise NotImplementedError("write your pallas kernel here")


# speedup vs baseline: 9.3224x; 9.3224x over previous
import functools

import jax
import jax.numpy as jnp
from jax import lax
from jax.experimental import pallas as pl
from jax.experimental.pallas import tpu as pltpu
from jax.experimental.pallas import tpu_sc as plsc

N = 10000
E = 320000
D = 128
DO = 16

NC = 2
NS = 16
NW = NC * NS

NPAD = 10240
RPT = NPAD // NS
CHUNK = 128
EPT = 10112
EPAD = EPT * NW
NCHUNK = EPT // CHUNK
PAD_DST = NPAD - 8

_sc_mesh = plsc.VectorSubcoreMesh(core_axis_name="c", subcore_axis_name="s")


@functools.partial(
    pl.kernel,
    out_type=jax.ShapeDtypeStruct((NC, NPAD), jnp.float32),
    mesh=_sc_mesh,
    scratch_types=[
        pltpu.VMEM((CHUNK,), jnp.int32),
        pltpu.VMEM((CHUNK,), jnp.float32),
        pltpu.VMEM_SHARED((NPAD,), jnp.float32),
    ],
)
def _deg_kernel(col_hbm, ones_hbm, z1_hbm, out_hbm, idx_v, ones_v, acc):
    c = lax.axis_index("c")
    s = lax.axis_index("s")
    pltpu.sync_copy(ones_hbm, ones_v)
    pltpu.sync_copy(z1_hbm.at[pl.ds(s * RPT, RPT)], acc.at[pl.ds(s * RPT, RPT)])
    plsc.subcore_barrier()
    base = (c * NS + s) * EPT

    def step(j, carry):
        pltpu.sync_copy(col_hbm.at[pl.ds(base + j * CHUNK, CHUNK)], idx_v)
        pltpu.sync_copy(ones_v, acc.at[idx_v], add=True)
        return carry

    lax.fori_loop(0, NCHUNK, step, 0)
    plsc.subcore_barrier()
    pltpu.sync_copy(acc.at[pl.ds(s * RPT, RPT)],
                    out_hbm.at[c, pl.ds(s * RPT, RPT)])


@functools.partial(
    pl.kernel,
    out_type=jax.ShapeDtypeStruct((NC, NPAD, D), jnp.float32),
    mesh=_sc_mesh,
    scratch_types=[
        pltpu.VMEM((CHUNK,), jnp.int32),
        pltpu.VMEM((CHUNK,), jnp.int32),
        pltpu.VMEM((CHUNK, D), jnp.float32),
        pltpu.VMEM_SHARED((NPAD, D), jnp.float32),
        pltpu.SemaphoreType.DMA,
    ],
)
def _agg_kernel(g_hbm, row_hbm, col_hbm, z2_hbm, out_hbm,
                ridx_v, cidx_v, gbuf, acc, sem):
    c = lax.axis_index("c")
    s = lax.axis_index("s")
    pltpu.sync_copy(z2_hbm.at[pl.ds(s * RPT, RPT)], acc.at[pl.ds(s * RPT, RPT)])
    plsc.subcore_barrier()
    base = (c * NS + s) * EPT

    def step(j, carry):
        pltpu.sync_copy(row_hbm.at[pl.ds(base + j * CHUNK, CHUNK)], ridx_v)
        pltpu.sync_copy(col_hbm.at[pl.ds(base + j * CHUNK, CHUNK)], cidx_v)
        pltpu.async_copy(g_hbm.at[ridx_v], gbuf, sem).wait()
        pltpu.sync_copy(gbuf, acc.at[cidx_v], add=True)
        return carry

    lax.fori_loop(0, NCHUNK, step, 0)
    plsc.subcore_barrier()
    pltpu.sync_copy(acc.at[pl.ds(s * RPT, RPT)],
                    out_hbm.at[c, pl.ds(s * RPT, RPT)])


BLK = 1000
_GRID = (N // BLK,)


def _k1_body(x_ref, w_ref, degc_ref, out_ref):
    dis = lax.rsqrt(degc_ref[...])
    out_ref[...] = dis * jnp.dot(x_ref[...], w_ref[...],
                                 preferred_element_type=jnp.float32)


_k1 = pl.pallas_call(
    _k1_body,
    grid=_GRID,
    in_specs=[
        pl.BlockSpec((BLK, D), lambda i: (i, 0)),
        pl.BlockSpec((D, D), lambda i: (0, 0)),
        pl.BlockSpec((BLK, 1), lambda i: (i, 0)),
    ],
    out_specs=pl.BlockSpec((BLK, D), lambda i: (i, 0)),
    out_shape=jax.ShapeDtypeStruct((N, D), jnp.float32),
)


def _k2_body(p_ref, g_ref, degc_ref, b_ref, w_ref, out_ref):
    dis = lax.rsqrt(degc_ref[...])
    p = p_ref[...]
    act = jnp.maximum(dis * (p[0] + p[1] + g_ref[...]) + b_ref[...], 0.0)
    out_ref[...] = dis * jnp.dot(act, w_ref[...],
                                 preferred_element_type=jnp.float32)


_k2 = pl.pallas_call(
    _k2_body,
    grid=_GRID,
    in_specs=[
        pl.BlockSpec((NC, BLK, D), lambda i: (0, i, 0)),
        pl.BlockSpec((BLK, D), lambda i: (i, 0)),
        pl.BlockSpec((BLK, 1), lambda i: (i, 0)),
        pl.BlockSpec((1, D), lambda i: (0, 0)),
        pl.BlockSpec((D, D), lambda i: (0, 0)),
    ],
    out_specs=pl.BlockSpec((BLK, D), lambda i: (i, 0)),
    out_shape=jax.ShapeDtypeStruct((N, D), jnp.float32),
)


def _k4_body(p_ref, g_ref, degc_ref, b_ref, wf1_ref, bf1_ref, wf2_ref,
             bf2_ref, out_ref):
    dis = lax.rsqrt(degc_ref[...])
    p = p_ref[...]
    act = jnp.maximum(dis * (p[0] + p[1] + g_ref[...]) + b_ref[...], 0.0)
    t = jnp.maximum(jnp.dot(act, wf1_ref[...],
                            preferred_element_type=jnp.float32) + bf1_ref[...],
                    0.0)
    out_ref[...] = jnp.dot(t, wf2_ref[...],
                           preferred_element_type=jnp.float32) + bf2_ref[...]


_k4 = pl.pallas_call(
    _k4_body,
    grid=_GRID,
    in_specs=[
        pl.BlockSpec((NC, BLK, D), lambda i: (0, i, 0)),
        pl.BlockSpec((BLK, D), lambda i: (i, 0)),
        pl.BlockSpec((BLK, 1), lambda i: (i, 0)),
        pl.BlockSpec((1, D), lambda i: (0, 0)),
        pl.BlockSpec((D, D), lambda i: (0, 0)),
        pl.BlockSpec((1, D), lambda i: (0, 0)),
        pl.BlockSpec((D, DO), lambda i: (0, 0)),
        pl.BlockSpec((1, DO), lambda i: (0, 0)),
    ],
    out_specs=pl.BlockSpec((BLK, DO), lambda i: (i, 0)),
    out_shape=jax.ShapeDtypeStruct((N, DO), jnp.float32),
)


def kernel(x, edge_index, W1, b1, W2, b2, W3, b3, Wf1, bf1, Wf2, bf2):
    row = edge_index[0]
    col = edge_index[1]
    pad = EPAD - E
    row_p = jnp.concatenate([row, jnp.zeros((pad,), row.dtype)])
    col_p = jnp.concatenate([col, jnp.full((pad,), PAD_DST, col.dtype)])
    ones_c = jnp.ones((CHUNK,), jnp.float32)
    z1 = jnp.zeros((NPAD,), jnp.float32)
    z2 = jnp.zeros((NPAD, D), jnp.float32)

    degp = _deg_kernel(col_p, ones_c, z1)
    degc = (1.0 + degp[0, :N] + degp[1, :N])[:, None]

    g1 = _k1(x, W1, degc)
    p1 = _agg_kernel(g1, row_p, col_p, z2)
    g2 = _k2(p1, g1, degc, b1.reshape(1, D), W2)
    p2 = _agg_kernel(g2, row_p, col_p, z2)
    g3 = _k2(p2, g2, degc, b2.reshape(1, D), W3)
    p3 = _agg_kernel(g3, row_p, col_p, z2)
    pred = _k4(p3, g3, degc, b3.reshape(1, D), Wf1, bf1.reshape(1, D),
               Wf2, bf2.reshape(1, DO))
    return pred
